# Initial kernel scaffold; baseline (speedup 1.0000x reference)
import jax
import jax.numpy as jnp
from jax.experimental import pallas as pl


def kernel(x, edge_index, edge_attr, params):
    raise NotImplementedError("write your pallas kernel here")


# speedup vs baseline: 3.2707x; 3.2707x over previous
import functools

import jax
import jax.numpy as jnp
from jax import lax
from jax.experimental import pallas as pl
from jax.experimental.pallas import tpu as pltpu
from jax.experimental.pallas import tpu_sc as plsc

NN = 50000
EE = 800000
DIN = 128
DE = 16
HH = 64
NEG = 0.2
NC = 2
NS = 16
NWK = NC * NS
EPT = EE // NWK
EPS = EE // NS

_MESH = plsc.VectorSubcoreMesh(core_axis_name="c", subcore_axis_name="s")

_F32 = jnp.float32
_I32 = jnp.int32

_CP = 3128
_CP_LAST = NN - (NS - 1) * _CP
_ZR = 200


def _fill_vec(ref, rows, cols, vec):
    nslot = cols // 16

    def body(i, _):
        r = i // nslot
        s = i % nslot
        ref[r, pl.ds(s * 16, 16)] = vec
        return 0

    lax.fori_loop(0, rows * nslot, body, 0)


def _zero_idx(ref, n):
    z = jnp.zeros((16,), _I32)
    for i in range(n // 16):
        ref[pl.ds(i * 16, 16)] = z


def _zero_spmem(acc, zb, sid):
    nchunk = NN // _ZR

    def body(j, _):
        c = sid + NS * j

        @pl.when(c < nchunk)
        def _():
            r0 = pl.multiple_of(c * _ZR, 8)
            pltpu.sync_copy(zb, acc.at[pl.ds(r0, _ZR)])
        return 0

    lax.fori_loop(0, (nchunk + NS - 1) // NS, body, 0)


def _copy_out_rows(acc, out_at, sid):
    @pl.when(sid < NS - 1)
    def _():
        r0 = pl.multiple_of(sid * _CP, 8)
        pltpu.sync_copy(acc.at[pl.ds(r0, _CP)], out_at.at[pl.ds(r0, _CP)])

    @pl.when(sid == NS - 1)
    def _():
        r0 = (NS - 1) * _CP
        pltpu.sync_copy(acc.at[pl.ds(r0, _CP_LAST)],
                        out_at.at[pl.ds(r0, _CP_LAST)])


@functools.partial(
    pl.kernel,
    out_type=(
        jax.ShapeDtypeStruct((NC, NN, DE), _F32),
        jax.ShapeDtypeStruct((NC, NN, DE), _F32),
    ),
    mesh=_MESH,
    compiler_params=pltpu.CompilerParams(needs_layout_passes=False, use_tc_tiling_on_sc=False),
    scratch_types=[
        pltpu.VMEM((128,), _I32),
        pltpu.VMEM((48,), _I32),
        pltpu.VMEM((128, DE), _F32),
        pltpu.VMEM((48, DE), _F32),
        pltpu.VMEM((_ZR, DE), _F32),
        pltpu.VMEM_SHARED((NN, DE), _F32),
    ],
)
def _pass0(dst_hbm, ea_hbm, outL, outD, idx, idxt, ea, eat, zb, acc):
    cid = lax.axis_index("c")
    sid = lax.axis_index("s")
    wid = sid * NC + cid
    base0 = wid * EPT
    zv = jnp.zeros((16,), _F32)
    ov = jnp.ones((16,), _F32)

    _fill_vec(zb, _ZR, DE, zv)
    _zero_idx(idxt, 48)
    pltpu.sync_copy(dst_hbm.at[pl.ds(base0 + 195 * 128, 40)],
                    idxt.at[pl.ds(0, 40)])

    _zero_spmem(acc, zb, sid)
    plsc.subcore_barrier()

    def chunk1(k, _):
        b = base0 + k * 128
        pltpu.sync_copy(dst_hbm.at[pl.ds(b, 128)], idx)
        pltpu.sync_copy(ea_hbm.at[pl.ds(b, 128)], ea)
        pltpu.sync_copy(ea, acc.at[idx], add=True)
        return 0
    lax.fori_loop(0, 195, chunk1, 0)

    _fill_vec(eat, 48, DE, zv)
    pltpu.sync_copy(ea_hbm.at[pl.ds(base0 + 195 * 128, 40)],
                    eat.at[pl.ds(0, 40)])
    pltpu.sync_copy(eat, acc.at[idxt], add=True)

    plsc.subcore_barrier()
    _copy_out_rows(acc, outL.at[cid], sid)
    plsc.subcore_barrier()

    _zero_spmem(acc, zb, sid)
    plsc.subcore_barrier()

    _fill_vec(ea, 128, DE, ov)

    def chunk2(k, _):
        b = base0 + k * 128
        pltpu.sync_copy(dst_hbm.at[pl.ds(b, 128)], idx)
        pltpu.sync_copy(ea, acc.at[idx], add=True)
        return 0
    lax.fori_loop(0, 195, chunk2, 0)

    _fill_vec(eat, 40, DE, ov)
    pltpu.sync_copy(eat, acc.at[idxt], add=True)

    plsc.subcore_barrier()
    _copy_out_rows(acc, outD.at[cid], sid)


@functools.partial(
    pl.kernel,
    out_type=(
        jax.ShapeDtypeStruct((4, EE, DE), _F32),
        jax.ShapeDtypeStruct((NC, NN, DE), _F32),
    ),
    mesh=_MESH,
    compiler_params=pltpu.CompilerParams(needs_layout_passes=False, use_tc_tiling_on_sc=False),
    scratch_types=[
        pltpu.VMEM((128,), _I32),
        pltpu.VMEM((128,), _I32),
        pltpu.VMEM((128, HH), _F32),
        pltpu.VMEM((128, HH), _F32),
        pltpu.VMEM((128, HH), _F32),
        pltpu.VMEM((128, DE), _F32),
        pltpu.VMEM((128, DE), _F32),
        pltpu.VMEM((128, DE), _F32),
        pltpu.VMEM((128, DE), _F32),
        pltpu.VMEM((128, DE), _F32),
        pltpu.VMEM((HH, 16), _F32),
        pltpu.VMEM((_ZR, DE), _F32),
        pltpu.VMEM_SHARED((NN, DE), _F32),
        pltpu.SemaphoreType.DMA,
    ],
)
def _pass1(src_hbm, dst_hbm, xl_hbm, xr_hbm, e_hbm, att_hbm,
           msg_out, den_out,
           idxs, idxd, bL, bR, bE, bM0, bM1, bM2, bM3, bXw,
           attb, zb, denS, sem):
    cid = lax.axis_index("c")
    sid = lax.axis_index("s")
    wid = sid * NC + cid
    base0 = wid * EPT

    pltpu.sync_copy(att_hbm, attb)
    _fill_vec(zb, _ZR, DE, jnp.zeros((16,), _F32))
    _zero_spmem(denS, zb, sid)
    plsc.subcore_barrier()

    iot = lax.iota(_I32, 16)
    rows_list = [iot + g * 16 for g in range(8)]
    bMs = [bM0, bM1, bM2, bM3]

    def _edge_chunk(b, real):
        pltpu.sync_copy(src_hbm.at[pl.ds(b, real)], idxs.at[pl.ds(0, real)])
        pltpu.sync_copy(dst_hbm.at[pl.ds(b, real)], idxd.at[pl.ds(0, real)])
        cpL = pltpu.async_copy(xl_hbm.at[idxs], bL, sem)
        cpR = pltpu.async_copy(xr_hbm.at[idxd], bR, sem)
        cpE = pltpu.async_copy(e_hbm.at[pl.ds(b, real)],
                               bE.at[pl.ds(0, real)], sem)
        cpL.wait()
        cpR.wait()
        cpE.wait()

        def dbody(d, accs):
            dv = jnp.full((16,), d, _I32)
            ad = plsc.load_gather(attb, [dv, iot])
            new = []
            for g in range(8):
                rows = rows_list[g]
                xld = plsc.load_gather(bL, [rows, dv])
                xrd = plsc.load_gather(bR, [rows, dv])
                ed = plsc.load_gather(bE, [rows, dv])
                z = xld + xrd + ed
                z = jnp.maximum(z, NEG * z)
                new.append(accs[g] + ad * z)
            return tuple(new)

        accs = lax.fori_loop(0, HH, dbody,
                             tuple(jnp.zeros((16,), _F32) for _ in range(8)))
        if real == 128:
            exs = [jnp.exp(a) for a in accs]
        else:
            exs = [jnp.where(rows_list[g] < real, jnp.exp(accs[g]), 0.0)
                   for g in range(8)]

        for q in range(4):
            bM = bMs[q]

            def mbody(d, _, q=q, bM=bM):
                dv = jnp.full((16,), d, _I32)
                for g in range(8):
                    rows = rows_list[g]
                    xld = plsc.load_gather(bL, [rows, dv + q * DE])
                    plsc.store_scatter(bM, [rows, dv], exs[g] * xld)
                return 0
            lax.fori_loop(0, DE, mbody, 0)

        def xbody(d, _):
            dv = jnp.full((16,), d, _I32)
            for g in range(8):
                plsc.store_scatter(bXw, [rows_list[g], dv], exs[g])
            return 0
        lax.fori_loop(0, DE, xbody, 0)

        pltpu.sync_copy(bXw, denS.at[idxd], add=True)
        for q in range(4):
            pltpu.sync_copy(bMs[q].at[pl.ds(0, real)],
                            msg_out.at[q, pl.ds(b, real)])

    def chunk(k, _):
        _edge_chunk(base0 + k * 128, 128)
        return 0
    lax.fori_loop(0, 195, chunk, 0)

    _zero_idx(idxs, 128)
    _zero_idx(idxd, 128)
    _edge_chunk(base0 + 195 * 128, 40)

    plsc.subcore_barrier()
    _copy_out_rows(denS, den_out.at[cid], sid)


@functools.partial(
    pl.kernel,
    out_type=jax.ShapeDtypeStruct((4, NN, DE), _F32),
    mesh=_MESH,
    compiler_params=pltpu.CompilerParams(needs_layout_passes=False, use_tc_tiling_on_sc=False),
    scratch_types=[
        pltpu.VMEM((128,), _I32),
        pltpu.VMEM((128, DE), _F32),
        pltpu.VMEM((_ZR, DE), _F32),
        pltpu.VMEM_SHARED((NN, DE), _F32),
    ],
)
def _pass2(dst_hbm, msg_hbm, acc_out, idx, buf, zb, accS):
    cid = lax.axis_index("c")
    sid = lax.axis_index("s")
    base0 = sid * EPS
    zv = jnp.zeros((16,), _F32)

    _fill_vec(zb, _ZR, DE, zv)

    for phase in range(2):
        q = cid * 2 + phase

        _zero_spmem(accS, zb, sid)
        plsc.subcore_barrier()

        def chunk(k, _, q=q):
            b = base0 + k * 128
            pltpu.sync_copy(dst_hbm.at[pl.ds(b, 128)], idx)
            pltpu.sync_copy(msg_hbm.at[q, pl.ds(b, 128)], buf)
            pltpu.sync_copy(buf, accS.at[idx], add=True)
            return 0
        lax.fori_loop(0, 390, chunk, 0)

        b = base0 + 390 * 128
        _zero_idx(idx, 128)
        for r in range(80, 128):
            buf[r, pl.ds(0, DE)] = zv
        pltpu.sync_copy(dst_hbm.at[pl.ds(b, 80)], idx.at[pl.ds(0, 80)])
        pltpu.sync_copy(msg_hbm.at[q, pl.ds(b, 80)], buf.at[pl.ds(0, 80)])
        pltpu.sync_copy(buf, accS.at[idx], add=True)

        plsc.subcore_barrier()
        _copy_out_rows(accS, acc_out.at[q], sid)
        if phase == 0:
            plsc.subcore_barrier()


_RB = 1000
_REB = 2000


def _prep_body(x, wpt, bp, lng, lnb, wlt, bl, wrt, br, wet, attr, sl, sd,
               xl_o, xr_o, ex_o, lm_o):
    h = jnp.dot(x[...], wpt[...], preferred_element_type=_F32) + bp[...]
    mu = jnp.mean(h, axis=-1, keepdims=True)
    var = jnp.mean((h - mu) ** 2, axis=-1, keepdims=True)
    hn = (h - mu) * lax.rsqrt(var + 1e-5) * lng[...] + lnb[...]
    xl = jnp.dot(hn, wlt[...], preferred_element_type=_F32) + bl[...]
    xr = jnp.dot(hn, wrt[...], preferred_element_type=_F32) + br[...]
    deg = sd[...][0, :, 0:1] + sd[...][1, :, 0:1]
    lm = (sl[...][0] + sl[...][1]) / jnp.maximum(deg, 1.0)
    el = jnp.dot(lm, wet[...], preferred_element_type=_F32)
    z = xl + xr + el
    z = jnp.maximum(z, NEG * z)
    logit = jnp.sum(z * attr[...], axis=-1, keepdims=True)
    xl_o[...] = xl
    xr_o[...] = xr
    ex_o[...] = jnp.exp(logit)
    lm_o[...] = lm


def _finish_prep_body(acc, den, exs, xl, bias, lng, lnb, wlt, bl, wrt, br,
                      wet, attr, lm, xl_o, xr_o, ex_o):
    a = acc[...]
    num = jnp.concatenate([a[0], a[1], a[2], a[3]], axis=-1) + exs[...] * xl[...]
    d = den[...][0, :, 0:1] + den[...][1, :, 0:1] + exs[...] + 1e-16
    h = jnp.maximum(num / d + bias[...], 0.0)
    mu = jnp.mean(h, axis=-1, keepdims=True)
    var = jnp.mean((h - mu) ** 2, axis=-1, keepdims=True)
    hn = (h - mu) * lax.rsqrt(var + 1e-5) * lng[...] + lnb[...]
    xl2 = jnp.dot(hn, wlt[...], preferred_element_type=_F32) + bl[...]
    xr2 = jnp.dot(hn, wrt[...], preferred_element_type=_F32) + br[...]
    el = jnp.dot(lm[...], wet[...], preferred_element_type=_F32)
    z = xl2 + xr2 + el
    z = jnp.maximum(z, NEG * z)
    logit = jnp.sum(z * attr[...], axis=-1, keepdims=True)
    xl_o[...] = xl2
    xr_o[...] = xr2
    ex_o[...] = jnp.exp(logit)


def _finish_cls_body(acc, den, exs, xl, bias, wct, bc, cls_o, h_o):
    a = acc[...]
    num = jnp.concatenate([a[0], a[1], a[2], a[3]], axis=-1) + exs[...] * xl[...]
    d = den[...][0, :, 0:1] + den[...][1, :, 0:1] + exs[...] + 1e-16
    h = jnp.maximum(num / d + bias[...], 0.0)
    h_o[...] = h
    cls_o[...] = jnp.dot(h, wct[...], preferred_element_type=_F32) + bc[...]


def _edge_e_body(ea, w1, w2, e1_o, e2_o):
    a = ea[...]
    e1_o[...] = jnp.dot(a, w1[...], preferred_element_type=_F32)
    e2_o[...] = jnp.dot(a, w2[...], preferred_element_type=_F32)


def _row_spec(shape):
    nd = len(shape)
    if nd == 2:
        return pl.BlockSpec((_RB, shape[1]), lambda i: (i, 0))
    return pl.BlockSpec((shape[0], _RB, shape[2]), lambda i: (0, i, 0))


def _const_spec(shape):
    return pl.BlockSpec(shape, lambda i: tuple(0 for _ in shape))


def _tc_edge_e(ea, wet1, wet2):
    return pl.pallas_call(
        _edge_e_body,
        grid=(EE // _REB,),
        in_specs=[pl.BlockSpec((_REB, DE), lambda i: (i, 0)),
                  _const_spec((DE, HH)), _const_spec((DE, HH))],
        out_specs=[pl.BlockSpec((_REB, HH), lambda i: (i, 0)),
                   pl.BlockSpec((_REB, HH), lambda i: (i, 0))],
        out_shape=[jax.ShapeDtypeStruct((EE, HH), _F32),
                   jax.ShapeDtypeStruct((EE, HH), _F32)],
    )(ea, wet1, wet2)


def _tc_prep(x, wpt, bp, lng, lnb, wlt, bl, wrt, br, wet, attr, sumL, sumD):
    return pl.pallas_call(
        _prep_body,
        grid=(NN // _RB,),
        in_specs=[pl.BlockSpec((_RB, DIN), lambda i: (i, 0)),
                  _const_spec((DIN, HH)), _const_spec((1, HH)),
                  _const_spec((1, HH)), _const_spec((1, HH)),
                  _const_spec((HH, HH)), _const_spec((1, HH)),
                  _const_spec((HH, HH)), _const_spec((1, HH)),
                  _const_spec((DE, HH)), _const_spec((1, HH)),
                  _row_spec((NC, NN, DE)), _row_spec((NC, NN, DE))],
        out_specs=[pl.BlockSpec((_RB, HH), lambda i: (i, 0)),
                   pl.BlockSpec((_RB, HH), lambda i: (i, 0)),
                   pl.BlockSpec((_RB, 1), lambda i: (i, 0)),
                   pl.BlockSpec((_RB, DE), lambda i: (i, 0))],
        out_shape=[jax.ShapeDtypeStruct((NN, HH), _F32),
                   jax.ShapeDtypeStruct((NN, HH), _F32),
                   jax.ShapeDtypeStruct((NN, 1), _F32),
                   jax.ShapeDtypeStruct((NN, DE), _F32)],
    )(x, wpt, bp, lng, lnb, wlt, bl, wrt, br, wet, attr, sumL, sumD)


def _tc_finish_prep(acc, den, exs, xl, bias, lng, lnb, wlt, bl, wrt, br,
                    wet, attr, lm):
    return pl.pallas_call(
        _finish_prep_body,
        grid=(NN // _RB,),
        in_specs=[_row_spec((4, NN, DE)), _row_spec((NC, NN, DE)),
                  pl.BlockSpec((_RB, 1), lambda i: (i, 0)),
                  pl.BlockSpec((_RB, HH), lambda i: (i, 0)),
                  _const_spec((1, HH)),
                  _const_spec((1, HH)), _const_spec((1, HH)),
                  _const_spec((HH, HH)), _const_spec((1, HH)),
                  _const_spec((HH, HH)), _const_spec((1, HH)),
                  _const_spec((DE, HH)), _const_spec((1, HH)),
                  pl.BlockSpec((_RB, DE), lambda i: (i, 0))],
        out_specs=[pl.BlockSpec((_RB, HH), lambda i: (i, 0)),
                   pl.BlockSpec((_RB, HH), lambda i: (i, 0)),
                   pl.BlockSpec((_RB, 1), lambda i: (i, 0))],
        out_shape=[jax.ShapeDtypeStruct((NN, HH), _F32),
                   jax.ShapeDtypeStruct((NN, HH), _F32),
                   jax.ShapeDtypeStruct((NN, 1), _F32)],
    )(acc, den, exs, xl, bias, lng, lnb, wlt, bl, wrt, br, wet, attr, lm)


def _tc_finish_cls(acc, den, exs, xl, bias, wct, bc):
    return pl.pallas_call(
        _finish_cls_body,
        grid=(NN // _RB,),
        in_specs=[_row_spec((4, NN, DE)), _row_spec((NC, NN, DE)),
                  pl.BlockSpec((_RB, 1), lambda i: (i, 0)),
                  pl.BlockSpec((_RB, HH), lambda i: (i, 0)),
                  _const_spec((1, HH)),
                  _const_spec((HH, 64)), _const_spec((1, 64))],
        out_specs=[pl.BlockSpec((_RB, 64), lambda i: (i, 0)),
                   pl.BlockSpec((_RB, HH), lambda i: (i, 0))],
        out_shape=[jax.ShapeDtypeStruct((NN, 64), _F32),
                   jax.ShapeDtypeStruct((NN, HH), _F32)],
    )(acc, den, exs, xl, bias, wct, bc)


def kernel(x, edge_index, edge_attr, params):
    src = edge_index[0]
    dst = edge_index[1]
    p1, p2 = params['layers']

    wpt = params['Wp'].T
    bp = params['bp'].reshape(1, HH)
    wct = params['Wc'].T
    bc = params['bc'].reshape(1, 64)

    def layer_mats(p):
        return (p['ln_g'].reshape(1, HH), p['ln_b'].reshape(1, HH),
                p['Wl'].T, p['bl'].reshape(1, HH),
                p['Wr'].T, p['br'].reshape(1, HH),
                p['We'].T, p['att'].reshape(1, HH),
                jnp.broadcast_to(p['att'].reshape(HH, 1), (HH, 16)),
                p['bias'].reshape(1, HH))

    (lng1, lnb1, wlt1, bl1, wrt1, br1, wet1, attr1, attv1, bias1) = layer_mats(p1)
    (lng2, lnb2, wlt2, bl2, wrt2, br2, wet2, attr2, attv2, bias2) = layer_mats(p2)

    sumL, sumD = _pass0(dst, edge_attr)
    e1, e2 = _tc_edge_e(edge_attr, wet1, wet2)
    xl1, xr1, exs1, lm = _tc_prep(x, wpt, bp, lng1, lnb1, wlt1, bl1, wrt1,
                                  br1, wet1, attr1, sumL, sumD)
    msg1, den1 = _pass1(src, dst, xl1, xr1, e1, attv1)
    acc1 = _pass2(dst, msg1)
    xl2, xr2, exs2 = _tc_finish_prep(acc1, den1, exs1, xl1, bias1, lng2,
                                     lnb2, wlt2, bl2, wrt2, br2, wet2,
                                     attr2, lm)
    msg2, den2 = _pass1(src, dst, xl2, xr2, e2, attv2)
    acc2 = _pass2(dst, msg2)
    cls, h = _tc_finish_cls(acc2, den2, exs2, xl2, bias2, wct, bc)
    return (cls, h)
